# baseline (speedup 1.0000x reference)
import functools

import numpy as np

import jax
import jax.numpy as jnp
from jax.experimental import pallas as pl
from jax.experimental.pallas import tpu as pltpu

_BM = 400
_BC = 1024
_F8 = jnp.float8_e4m3fn


def _layer1_body(adj_ref, x_ref, w1_ref, b1_ref, w2_ref,
                 s2_out_ref, zpart_ref, q1_ref, q2_ref,
                 s1_ref, s2sc_ref, *, nq1, nend):
    i = pl.program_id(0)
    c = pl.program_id(1)
    end0 = nq1 * _BC

    @pl.when(c == 0)
    def _():
        @pl.when(i == 0)
        def _():
            s1_ref[...] = jnp.dot(x_ref[...], w1_ref[...],
                                  preferred_element_type=jnp.float32)
            s2sc_ref[...] = jnp.zeros_like(s2sc_ref)

        a = adj_ref[...]
        zpart_ref[...] = jnp.dot(a, s2sc_ref[...],
                                 preferred_element_type=jnp.float32)
        acc = jnp.dot(a, s1_ref[...], preferred_element_type=jnp.float32)
        h = jnp.maximum(acc + b1_ref[...], 0.0)
        s2t = jnp.dot(h, w2_ref[...], preferred_element_type=jnp.float32)
        s2_out_ref[...] = s2t
        s2sc_ref[pl.ds(i * _BM, _BM), :] = s2t

        a_end = adj_ref[:, pl.ds(end0, nend)]
        colq2 = jax.lax.broadcasted_iota(jnp.int32, (_BM, nend), 1)
        q2_ref[...] = jnp.where(colq2 >= _BM * i - end0, a_end, 0.0).astype(_F8)

    @pl.when(c >= (_BM * i) // _BC)
    def _():
        a_sl = adj_ref[:, pl.ds(c * _BC, _BC)]
        col = jax.lax.broadcasted_iota(jnp.int32, (_BM, _BC), 1)
        q1_ref[...] = jnp.where(col >= _BM * i - _BC * c, a_sl, 0.0).astype(_F8)


def _layer2_body(imap_ref, cmap_ref, first_ref, slot_ref, q1_ref, q2_ref,
                 s2_ref, zpart_ref, b2_ref, o_ref, s2q_ref, scale_ref,
                 *, nq1, nend):
    m = pl.program_id(0)
    c = cmap_ref[m]
    end0 = nq1 * _BC

    @pl.when(m == 0)
    def _():
        s2 = s2_ref[...]
        mx = jnp.maximum(jnp.max(jnp.abs(s2)), 1e-30)
        s2q_ref[...] = (s2 * (448.0 / mx)).astype(_F8)
        scale_ref[0] = mx * (1.0 / 448.0)

    @pl.when(c < nq1)
    def _():
        contrib = jnp.dot(q1_ref[...], s2q_ref[pl.ds(c * _BC, _BC), :],
                          preferred_element_type=jnp.float32)

        @pl.when(first_ref[m] == 1)
        def _():
            o_ref[...] = contrib

        @pl.when(first_ref[m] == 0)
        def _():
            o_ref[...] += contrib

    @pl.when(c == nq1)
    def _():
        contrib = jnp.dot(q2_ref[...], s2q_ref[pl.ds(end0, nend), :],
                          preferred_element_type=jnp.float32)

        @pl.when(first_ref[m] == 1)
        def _():
            o_ref[...] = contrib

        @pl.when(first_ref[m] == 0)
        def _():
            o_ref[...] += contrib

        z = o_ref[...] * scale_ref[0] + zpart_ref[...] + b2_ref[...]
        zm = z - jnp.max(z, axis=1, keepdims=True)
        lse = jnp.log(jnp.sum(jnp.exp(zm), axis=1, keepdims=True))
        o_ref[...] = zm - lse


@jax.jit
def kernel(x, adj, W1, b1, W2, b2):
    n, nfeat = x.shape
    nhid = W1.shape[1]
    nclass = W2.shape[1]
    ti = n // _BM
    nq1 = (n - 1) // _BC
    nend = n - nq1 * _BC
    b1r = b1.reshape(1, nhid)
    b2r = b2.reshape(1, nclass)

    full = lambda i, c: (0, 0)
    row = lambda i, c: (i, 0)

    def q1_map(i, c, _ti=ti, _nq1=nq1):
        cc = jnp.minimum(jnp.maximum(c, (_BM * i) // _BC), _nq1 - 1)
        return (cc * _ti + i, 0)

    s2_out, zpart, q1, q2 = pl.pallas_call(
        functools.partial(_layer1_body, nq1=nq1, nend=nend),
        grid=(ti, nq1),
        in_specs=[
            pl.BlockSpec((_BM, n), row),
            pl.BlockSpec((n, nfeat), full),
            pl.BlockSpec((nfeat, nhid), full),
            pl.BlockSpec((1, nhid), full),
            pl.BlockSpec((nhid, nclass), full),
        ],
        out_specs=[
            pl.BlockSpec((_BM, nclass), row),
            pl.BlockSpec((_BM, nclass), row),
            pl.BlockSpec((_BM, _BC), q1_map),
            pl.BlockSpec((_BM, nend), row),
        ],
        out_shape=[
            jax.ShapeDtypeStruct((n, nclass), jnp.float32),
            jax.ShapeDtypeStruct((n, nclass), jnp.float32),
            jax.ShapeDtypeStruct((nq1 * ti * _BM, _BC), _F8),
            jax.ShapeDtypeStruct((n, nend), _F8),
        ],
        scratch_shapes=[
            pltpu.VMEM((n, nhid), jnp.float32),
            pltpu.VMEM((n, nclass), jnp.float32),
        ],
        compiler_params=pltpu.CompilerParams(
            dimension_semantics=("arbitrary", "arbitrary")),
    )(adj, x, W1, b1r, W2)

    imap, cmap, first, q1slot = [], [], [], []
    prev_slot = 0
    for i in range(ti):
        c0 = min((_BM * i) // _BC, nq1)
        for j, c in enumerate(list(range(c0, nq1)) + [nq1]):
            imap.append(i)
            cmap.append(c)
            first.append(1 if j == 0 else 0)
            if c < nq1:
                prev_slot = c * ti + i
            q1slot.append(prev_slot)

    out = pl.pallas_call(
        functools.partial(_layer2_body, nq1=nq1, nend=nend),
        grid_spec=pltpu.PrefetchScalarGridSpec(
            num_scalar_prefetch=4,
            grid=(len(imap),),
            in_specs=[
                pl.BlockSpec((_BM, _BC),
                             lambda m, im, cm, fr, sl: (sl[m], 0)),
                pl.BlockSpec((_BM, nend),
                             lambda m, im, cm, fr, sl: (im[m], 0)),
                pl.BlockSpec((n, nclass),
                             lambda m, im, cm, fr, sl: (0, 0)),
                pl.BlockSpec((_BM, nclass),
                             lambda m, im, cm, fr, sl: (im[m], 0)),
                pl.BlockSpec((1, nclass),
                             lambda m, im, cm, fr, sl: (0, 0)),
            ],
            out_specs=pl.BlockSpec((_BM, nclass),
                                   lambda m, im, cm, fr, sl: (im[m], 0)),
            scratch_shapes=[
                pltpu.VMEM((n, nclass), _F8),
                pltpu.SMEM((1,), jnp.float32),
            ],
        ),
        out_shape=jax.ShapeDtypeStruct((n, nclass), jnp.float32),
        compiler_params=pltpu.CompilerParams(
            dimension_semantics=("arbitrary",)),
    )(jnp.asarray(np.asarray(imap, np.int32)),
      jnp.asarray(np.asarray(cmap, np.int32)),
      jnp.asarray(np.asarray(first, np.int32)),
      jnp.asarray(np.asarray(q1slot, np.int32)),
      q1, q2, s2_out, zpart, b2r)

    return out


# speedup vs baseline: 1.5618x; 1.5618x over previous
import functools

import numpy as np

import jax
import jax.numpy as jnp
from jax.experimental import pallas as pl
from jax.experimental.pallas import tpu as pltpu

_BM = 400
_BC = 1024
_F8 = jnp.float8_e4m3fn


def _layer1_body(adj_ref, x_ref, w1_ref, b1_ref, w2_ref,
                 s2_out_ref, zpart_ref, q1_ref, q2_ref,
                 s1_ref, s2sc_ref, *, nq1, nend):
    i = pl.program_id(0)
    end0 = nq1 * _BC

    @pl.when(i == 0)
    def _():
        s1_ref[...] = jnp.dot(x_ref[...], w1_ref[...],
                              preferred_element_type=jnp.float32)
        s2sc_ref[...] = jnp.zeros_like(s2sc_ref)

    a = adj_ref[...]
    zpart_ref[...] = jnp.dot(a, s2sc_ref[...],
                             preferred_element_type=jnp.float32)
    acc = jnp.dot(a, s1_ref[...], preferred_element_type=jnp.float32)
    h = jnp.maximum(acc + b1_ref[...], 0.0)
    s2t = jnp.dot(h, w2_ref[...], preferred_element_type=jnp.float32)
    s2_out_ref[...] = s2t
    s2sc_ref[pl.ds(i * _BM, _BM), :] = s2t

    q1_ref[...] = a[:, :end0].astype(_F8)
    c0 = jnp.minimum((_BM * i) // _BC, nq1 - 1)
    start = c0 * _BC
    a_b = adj_ref[:, pl.ds(start, _BC)]
    col = jax.lax.broadcasted_iota(jnp.int32, (_BM, _BC), 1)
    q1_ref[:, pl.ds(start, _BC)] = jnp.where(
        col >= _BM * i - start, a_b, 0.0).astype(_F8)

    a_end = adj_ref[:, pl.ds(end0, nend)]
    colq2 = jax.lax.broadcasted_iota(jnp.int32, (_BM, nend), 1)
    q2_ref[...] = jnp.where(colq2 >= _BM * i - end0, a_end, 0.0).astype(_F8)


def _layer2_body(imap_ref, cmap_ref, first_ref, q1_ref, q2_ref,
                 s2_ref, zpart_ref, b2_ref, o_ref, s2q_ref, scale_ref,
                 *, nq1, nend):
    m = pl.program_id(0)
    c = cmap_ref[m]
    end0 = nq1 * _BC

    @pl.when(m == 0)
    def _():
        s2 = s2_ref[...]
        mx = jnp.maximum(jnp.max(jnp.abs(s2)), 1e-30)
        s2q_ref[...] = (s2 * (448.0 / mx)).astype(_F8)
        scale_ref[0] = mx * (1.0 / 448.0)

    @pl.when(c < nq1)
    def _():
        contrib = jnp.dot(q1_ref[...], s2q_ref[pl.ds(c * _BC, _BC), :],
                          preferred_element_type=jnp.float32)

        @pl.when(first_ref[m] == 1)
        def _():
            o_ref[...] = contrib

        @pl.when(first_ref[m] == 0)
        def _():
            o_ref[...] += contrib

    @pl.when(c == nq1)
    def _():
        contrib = jnp.dot(q2_ref[...], s2q_ref[pl.ds(end0, nend), :],
                          preferred_element_type=jnp.float32)

        @pl.when(first_ref[m] == 1)
        def _():
            o_ref[...] = contrib

        @pl.when(first_ref[m] == 0)
        def _():
            o_ref[...] += contrib

        z = o_ref[...] * scale_ref[0] + zpart_ref[...] + b2_ref[...]
        zm = z - jnp.max(z, axis=1, keepdims=True)
        lse = jnp.log(jnp.sum(jnp.exp(zm), axis=1, keepdims=True))
        o_ref[...] = zm - lse


@jax.jit
def kernel(x, adj, W1, b1, W2, b2):
    n, nfeat = x.shape
    nhid = W1.shape[1]
    nclass = W2.shape[1]
    ti = n // _BM
    nq1 = (n - 1) // _BC
    nend = n - nq1 * _BC
    b1r = b1.reshape(1, nhid)
    b2r = b2.reshape(1, nclass)

    full = lambda i: (0, 0)
    row = lambda i: (i, 0)

    s2_out, zpart, q1, q2 = pl.pallas_call(
        functools.partial(_layer1_body, nq1=nq1, nend=nend),
        grid=(ti,),
        in_specs=[
            pl.BlockSpec((_BM, n), row),
            pl.BlockSpec((n, nfeat), full),
            pl.BlockSpec((nfeat, nhid), full),
            pl.BlockSpec((1, nhid), full),
            pl.BlockSpec((nhid, nclass), full),
        ],
        out_specs=[
            pl.BlockSpec((_BM, nclass), row),
            pl.BlockSpec((_BM, nclass), row),
            pl.BlockSpec((_BM, nq1 * _BC), row),
            pl.BlockSpec((_BM, nend), row),
        ],
        out_shape=[
            jax.ShapeDtypeStruct((n, nclass), jnp.float32),
            jax.ShapeDtypeStruct((n, nclass), jnp.float32),
            jax.ShapeDtypeStruct((n, nq1 * _BC), _F8),
            jax.ShapeDtypeStruct((n, nend), _F8),
        ],
        scratch_shapes=[
            pltpu.VMEM((n, nhid), jnp.float32),
            pltpu.VMEM((n, nclass), jnp.float32),
        ],
        compiler_params=pltpu.CompilerParams(
            dimension_semantics=("arbitrary",)),
    )(adj, x, W1, b1r, W2)

    imap, cmap, first = [], [], []
    for i in range(ti):
        c0 = min((_BM * i) // _BC, nq1)
        for j, c in enumerate(list(range(c0, nq1)) + [nq1]):
            imap.append(i)
            cmap.append(c)
            first.append(1 if j == 0 else 0)

    out = pl.pallas_call(
        functools.partial(_layer2_body, nq1=nq1, nend=nend),
        grid_spec=pltpu.PrefetchScalarGridSpec(
            num_scalar_prefetch=3,
            grid=(len(imap),),
            in_specs=[
                pl.BlockSpec((_BM, _BC),
                             lambda m, im, cm, fr, _n=nq1: (
                                 im[m], jnp.minimum(cm[m], _n - 1))),
                pl.BlockSpec((_BM, nend),
                             lambda m, im, cm, fr: (im[m], 0)),
                pl.BlockSpec((n, nclass),
                             lambda m, im, cm, fr: (0, 0)),
                pl.BlockSpec((_BM, nclass),
                             lambda m, im, cm, fr: (im[m], 0)),
                pl.BlockSpec((1, nclass),
                             lambda m, im, cm, fr: (0, 0)),
            ],
            out_specs=pl.BlockSpec((_BM, nclass),
                                   lambda m, im, cm, fr: (im[m], 0)),
            scratch_shapes=[
                pltpu.VMEM((n, nclass), _F8),
                pltpu.SMEM((1,), jnp.float32),
            ],
        ),
        out_shape=jax.ShapeDtypeStruct((n, nclass), jnp.float32),
        compiler_params=pltpu.CompilerParams(
            dimension_semantics=("arbitrary",)),
    )(jnp.asarray(np.asarray(imap, np.int32)),
      jnp.asarray(np.asarray(cmap, np.int32)),
      jnp.asarray(np.asarray(first, np.int32)),
      q1, q2, s2_out, zpart, b2r)

    return out
